# baseline (speedup 1.0000x reference)
import functools
import jax
import jax.numpy as jnp
from jax import lax
from jax.experimental import pallas as pl
from jax.experimental.pallas import tpu as pltpu
from jax.experimental.pallas import tpu_sc as plsc

NC = 2
NS = 16
LANES = 16


def _sc_mesh():
    return plsc.VectorSubcoreMesh(core_axis_name="c", subcore_axis_name="s",
                                  num_cores=NC, num_subcores=NS)


def _node_span(n_nodes):
    return (((n_nodes + NS - 1) // NS) + LANES - 1) // LANES * LANES


def _make_deg_kernel(n_pad, n_chunks_pad):
    NW = NC * NS
    n_fire = n_chunks_pad // NW
    span = n_pad // NS

    @functools.partial(
        pl.kernel,
        out_type=jax.ShapeDtypeStruct((NC, n_pad), jnp.float32),
        mesh=_sc_mesh(),
        scratch_types=[
            pltpu.VMEM_SHARED((n_pad,), jnp.float32),
            pltpu.VMEM((span,), jnp.float32),
            pltpu.VMEM((128,), jnp.float32),
            pltpu.VMEM((n_fire, 1, 128), jnp.int32),
            pltpu.SemaphoreType.DMA,
        ],
    )
    def deg_kernel(dst_hbm, degp_hbm, deg_sh, zbuf, ones_v, idxb, sem):
        c = lax.axis_index("c")
        s = lax.axis_index("s")
        for k in range(span // LANES):
            zbuf[pl.ds(k * LANES, LANES)] = jnp.zeros((LANES,), jnp.float32)
        for k in range(128 // LANES):
            ones_v[pl.ds(k * LANES, LANES)] = jnp.ones((LANES,), jnp.float32)
        w = c * NS + s
        lo = w * n_fire
        pltpu.sync_copy(dst_hbm.at[pl.ds(lo, n_fire)], idxb)
        pltpu.sync_copy(zbuf, deg_sh.at[pl.ds(span * s, span)])
        plsc.subcore_barrier()

        def fire(t, _):
            pltpu.make_async_copy(ones_v, deg_sh.at[idxb.at[t, 0]], sem
                                  ).start(add=True)
            return ()

        def drain(t, _):
            pltpu.make_async_copy(ones_v, deg_sh.at[idxb.at[0, 0]], sem).wait()
            return ()

        lax.fori_loop(0, n_fire, fire, ())
        lax.fori_loop(0, n_fire, drain, ())
        plsc.subcore_barrier()

        pltpu.sync_copy(deg_sh.at[pl.ds(span * s, span)],
                        degp_hbm.at[c, pl.ds(span * s, span)])

    return deg_kernel


def _tc_scale_matmul(x, W, degp, n_pad, blk):
    n, d_in = x.shape
    d_out = W.shape[1]
    dh = d_out // NC

    def body(x_ref, w_ref, degp_ref, o_ref):
        deg = degp_ref[0][:, 0:1] + degp_ref[1][:, 0:1] + 1.0
        dinv = jnp.where(deg > 0, lax.rsqrt(deg), 0.0)
        h = jnp.dot(x_ref[...], w_ref[...], preferred_element_type=jnp.float32)
        g = h * dinv
        for cc in range(NC):
            o_ref[cc] = g[:, cc * dh:(cc + 1) * dh]

    return pl.pallas_call(
        body,
        grid=(n // blk,),
        in_specs=[
            pl.BlockSpec((blk, d_in), lambda i: (i, 0)),
            pl.BlockSpec((d_in, d_out), lambda i: (0, 0)),
            pl.BlockSpec((NC, blk, 1), lambda i: (0, i, 0)),
        ],
        out_specs=pl.BlockSpec((NC, blk, dh), lambda i: (0, i, 0)),
        out_shape=jax.ShapeDtypeStruct((NC, n_pad, dh), jnp.float32),
    )(x, W, degp)


def _make_agg_kernel(n_pad, n_chunks_pad, dh):
    nch = n_chunks_pad // NS
    assert nch % 2 == 0
    pairs = nch // 2
    span = n_pad // NS

    @functools.partial(
        pl.kernel,
        out_type=jax.ShapeDtypeStruct((NC, n_pad, dh), jnp.float32),
        mesh=_sc_mesh(),
        scratch_types=[
            pltpu.VMEM_SHARED((n_pad, dh), jnp.float32),
            pltpu.VMEM((128, dh), jnp.float32),
            pltpu.VMEM((128, dh), jnp.float32),
            pltpu.VMEM((nch, 1, 128), jnp.int32),
            pltpu.VMEM((128,), jnp.int32),
            pltpu.VMEM((128,), jnp.int32),
            pltpu.SemaphoreType.DMA,
            pltpu.SemaphoreType.DMA,
            pltpu.SemaphoreType.DMA,
            pltpu.SemaphoreType.DMA,
            pltpu.SemaphoreType.DMA,
            pltpu.SemaphoreType.DMA,
        ],
    )
    def agg_kernel(g_hbm, src_hbm, dst_hbm, acc_hbm,
                   acc_sh, buf_a, buf_b, sidx, didx_a, didx_b,
                   sg_a, sg_b, ss_a, ss_b, sd_a, sd_b):
        c = lax.axis_index("c")
        s = lax.axis_index("s")
        lo = s * nch
        ebase = s * (nch * 128)

        pltpu.sync_copy(src_hbm.at[c, pl.ds(lo, nch)], sidx)

        row0 = s * span
        pltpu.sync_copy(g_hbm.at[pl.ds(c * n_pad + row0, span)],
                        acc_sh.at[pl.ds(row0, span)])
        plsc.subcore_barrier()

        def gather(j, buf, sem):
            pltpu.make_async_copy(g_hbm.at[sidx.at[j, 0]], buf, sem).start()

        def wait_gather(buf, sem):
            pltpu.make_async_copy(g_hbm.at[sidx.at[0, 0]], buf, sem).wait()

        def load_didx(j, didx, sem):
            pltpu.make_async_copy(dst_hbm.at[pl.ds(ebase + j * 128, 128)],
                                  didx, sem).start()

        def wait_didx(didx, sem):
            pltpu.make_async_copy(dst_hbm.at[pl.ds(0, 128)], didx, sem).wait()

        def scat(buf, didx, sem):
            pltpu.make_async_copy(buf, acc_sh.at[didx], sem).start(add=True)

        def wait_scat(buf, didx, sem):
            pltpu.make_async_copy(buf, acc_sh.at[didx], sem).wait()

        load_didx(0, didx_a, sd_a)
        gather(0, buf_a, sg_a)

        def pair(t, _):
            wait_gather(buf_a, sg_a)
            wait_didx(didx_a, sd_a)
            @pl.when(t > 0)
            def _():
                wait_scat(buf_b, didx_b, ss_b)
            load_didx(2 * t + 1, didx_b, sd_b)
            gather(2 * t + 1, buf_b, sg_b)
            scat(buf_a, didx_a, ss_a)
            wait_gather(buf_b, sg_b)
            wait_didx(didx_b, sd_b)
            wait_scat(buf_a, didx_a, ss_a)
            @pl.when(t < pairs - 1)
            def _():
                load_didx(2 * t + 2, didx_a, sd_a)
                gather(2 * t + 2, buf_a, sg_a)
            scat(buf_b, didx_b, ss_b)
            return ()

        lax.fori_loop(0, pairs, pair, ())
        wait_scat(buf_b, didx_b, ss_b)
        plsc.subcore_barrier()

        pltpu.sync_copy(acc_sh.at[pl.ds(row0, span)],
                        acc_hbm.at[c, pl.ds(row0, span)])

    return agg_kernel


def _tc_pool(acc2, degp, ctb2, b2, n, n_types, blk):
    dh = acc2.shape[2]
    d = NC * dh
    nk = n // blk

    def body(a_ref, degp_ref, t_ref, b_ref, o_ref, acc, cnt):
        k = pl.program_id(0)
        deg = degp_ref[0][:, 0:1] + degp_ref[1][:, 0:1] + 1.0
        dinv = jnp.where(deg > 0, lax.rsqrt(deg), 0.0)
        a = jnp.concatenate([a_ref[0], a_ref[1]], axis=1)
        r = jnp.maximum(a * dinv + b_ref[...], 0.0)
        tids = lax.broadcasted_iota(jnp.int32, (blk, n_types), 1)
        m = (t_ref[...] == tids).astype(jnp.float32)
        part = lax.dot_general(m, r, (((0,), (0,)), ((), ())),
                               preferred_element_type=jnp.float32,
                               precision=lax.Precision.HIGHEST)
        cpart = lax.dot_general(m, jnp.ones((blk, 1), jnp.float32),
                                (((0,), (0,)), ((), ())),
                                preferred_element_type=jnp.float32,
                                precision=lax.Precision.HIGHEST)

        @pl.when(k == 0)
        def _():
            acc[...] = jnp.zeros_like(acc)
            cnt[...] = jnp.zeros_like(cnt)

        acc[...] += part
        cnt[...] += cpart

        @pl.when(k == nk - 1)
        def _():
            o_ref[...] = acc[...] / jnp.maximum(cnt[...], 1.0)

    return pl.pallas_call(
        body,
        grid=(nk,),
        in_specs=[
            pl.BlockSpec((NC, blk, dh), lambda i: (0, i, 0)),
            pl.BlockSpec((NC, blk, 1), lambda i: (0, i, 0)),
            pl.BlockSpec((blk, 1), lambda i: (i, 0)),
            pl.BlockSpec((1, d), lambda i: (0, 0)),
        ],
        out_specs=pl.BlockSpec((n_types, d), lambda i: (0, 0)),
        out_shape=jax.ShapeDtypeStruct((n_types, d), jnp.float32),
        scratch_shapes=[
            pltpu.VMEM((n_types, d), jnp.float32),
            pltpu.VMEM((n_types, 1), jnp.float32),
        ],
    )(acc2, degp, ctb2, b2)


def kernel(x, edge_index, batch, cell_type_batch, W, b):
    n, d_in = x.shape
    d_out = W.shape[1]
    dh = d_out // NC
    e = edge_index.shape[1]
    n_types = 100

    span = _node_span(n)
    n_pad = span * NS
    chunk_e = 128 * NC * NS
    e_pad = (e + chunk_e - 1) // chunk_e * chunk_e
    n_chunks_pad = e_pad // 128

    src = edge_index[0]
    dst = edge_index[1]
    src_p = jnp.concatenate([src, jnp.zeros((e_pad - e,), jnp.int32)])
    dst_p = jnp.concatenate([dst, jnp.full((e_pad - e,), n_pad - 1, jnp.int32)])
    src3 = jnp.stack([src_p, src_p + n_pad]).reshape(NC, n_chunks_pad, 1, 128)
    dst3 = dst_p.reshape(n_chunks_pad, 1, 128)

    degp2 = _make_deg_kernel(n_pad, n_chunks_pad)(dst3)
    degp = degp2.reshape(NC, n_pad, 1)

    g2 = _tc_scale_matmul(x, W, degp, n_pad, blk=1000)
    g_flat = g2.reshape(NC * n_pad, dh)
    acc2 = _make_agg_kernel(n_pad, n_chunks_pad, dh)(g_flat, src3, dst_p)

    pooled = _tc_pool(acc2, degp,
                      cell_type_batch.reshape(n, 1).astype(jnp.int32),
                      b.reshape(1, d_out), n, n_types, blk=1000)
    return pooled


# speedup vs baseline: 1.0019x; 1.0019x over previous
import functools
import jax
import jax.numpy as jnp
from jax import lax
from jax.experimental import pallas as pl
from jax.experimental.pallas import tpu as pltpu
from jax.experimental.pallas import tpu_sc as plsc

NC = 2
NS = 16
LANES = 16


def _sc_mesh():
    return plsc.VectorSubcoreMesh(core_axis_name="c", subcore_axis_name="s",
                                  num_cores=NC, num_subcores=NS)


def _node_span(n_nodes):
    return (((n_nodes + NS - 1) // NS) + LANES - 1) // LANES * LANES


def _make_deg_kernel(n_pad, n_chunks_pad):
    NW = NC * NS
    n_fire = n_chunks_pad // NW
    span = n_pad // NS

    @functools.partial(
        pl.kernel,
        out_type=jax.ShapeDtypeStruct((NC, n_pad), jnp.float32),
        mesh=_sc_mesh(),
        scratch_types=[
            pltpu.VMEM_SHARED((n_pad,), jnp.float32),
            pltpu.VMEM((span,), jnp.float32),
            pltpu.VMEM((128,), jnp.float32),
            pltpu.VMEM((n_fire, 1, 128), jnp.int32),
            pltpu.SemaphoreType.DMA,
        ],
    )
    def deg_kernel(dst_hbm, degp_hbm, deg_sh, zbuf, ones_v, idxb, sem):
        c = lax.axis_index("c")
        s = lax.axis_index("s")
        for k in range(span // LANES):
            zbuf[pl.ds(k * LANES, LANES)] = jnp.zeros((LANES,), jnp.float32)
        for k in range(128 // LANES):
            ones_v[pl.ds(k * LANES, LANES)] = jnp.ones((LANES,), jnp.float32)
        w = c * NS + s
        lo = w * n_fire
        pltpu.sync_copy(dst_hbm.at[pl.ds(lo, n_fire)], idxb)
        pltpu.sync_copy(zbuf, deg_sh.at[pl.ds(span * s, span)])
        plsc.subcore_barrier()

        def fire(t, _):
            pltpu.make_async_copy(ones_v, deg_sh.at[idxb.at[t, 0]], sem
                                  ).start(add=True)
            return ()

        def drain(t, _):
            pltpu.make_async_copy(ones_v, deg_sh.at[idxb.at[0, 0]], sem).wait()
            return ()

        lax.fori_loop(0, n_fire, fire, ())
        lax.fori_loop(0, n_fire, drain, ())
        plsc.subcore_barrier()

        pltpu.sync_copy(deg_sh.at[pl.ds(span * s, span)],
                        degp_hbm.at[c, pl.ds(span * s, span)])

    return deg_kernel


def _tc_scale_matmul(x, W, degp, n_pad, blk):
    n, d_in = x.shape
    d_out = W.shape[1]
    dh = d_out // NC

    def body(x_ref, w_ref, degp_ref, o_ref):
        deg = degp_ref[0][:, 0:1] + degp_ref[1][:, 0:1] + 1.0
        dinv = jnp.where(deg > 0, lax.rsqrt(deg), 0.0)
        h = jnp.dot(x_ref[...], w_ref[...], preferred_element_type=jnp.float32)
        g = h * dinv
        for cc in range(NC):
            o_ref[cc] = g[:, cc * dh:(cc + 1) * dh]

    return pl.pallas_call(
        body,
        grid=(n // blk,),
        in_specs=[
            pl.BlockSpec((blk, d_in), lambda i: (i, 0)),
            pl.BlockSpec((d_in, d_out), lambda i: (0, 0)),
            pl.BlockSpec((NC, blk, 1), lambda i: (0, i, 0)),
        ],
        out_specs=pl.BlockSpec((NC, blk, dh), lambda i: (0, i, 0)),
        out_shape=jax.ShapeDtypeStruct((NC, n_pad, dh), jnp.float32),
    )(x, W, degp)


def _make_agg_kernel(n_pad, n_chunks_pad, dh):
    nch = n_chunks_pad // NS
    assert nch % 2 == 0
    pairs = nch // 2
    span = n_pad // NS

    @functools.partial(
        pl.kernel,
        out_type=jax.ShapeDtypeStruct((NC, n_pad, dh), jnp.float32),
        mesh=_sc_mesh(),
        scratch_types=[
            pltpu.VMEM_SHARED((n_pad, dh), jnp.float32),
            pltpu.VMEM((128, dh), jnp.float32),
            pltpu.VMEM((128, dh), jnp.float32),
            pltpu.VMEM((nch * 128,), jnp.int32),
            pltpu.VMEM((128,), jnp.int32),
            pltpu.VMEM((128,), jnp.int32),
            pltpu.SemaphoreType.DMA,
            pltpu.SemaphoreType.DMA,
            pltpu.SemaphoreType.DMA,
            pltpu.SemaphoreType.DMA,
            pltpu.SemaphoreType.DMA,
            pltpu.SemaphoreType.DMA,
        ],
    )
    def agg_kernel(g_hbm, src_hbm, dst_hbm, acc_hbm,
                   acc_sh, buf_a, buf_b, sidx, didx_a, didx_b,
                   sg_a, sg_b, ss_a, ss_b, sd_a, sd_b):
        c = lax.axis_index("c")
        s = lax.axis_index("s")
        lo = s * nch
        ebase = s * (nch * 128)

        pltpu.sync_copy(src_hbm.at[c, pl.ds(ebase, nch * 128)], sidx)

        row0 = s * span
        pltpu.sync_copy(g_hbm.at[pl.ds(c * n_pad + row0, span)],
                        acc_sh.at[pl.ds(row0, span)])
        plsc.subcore_barrier()

        QG = 4
        sub = 128 // QG

        def gather(j, buf, sem):
            for q in range(QG):
                pltpu.make_async_copy(
                    g_hbm.at[sidx.at[pl.ds(j * 128 + q * sub, sub)]],
                    buf.at[pl.ds(q * sub, sub)], sem).start()

        def wait_gather(buf, sem):
            for q in range(QG):
                pltpu.make_async_copy(g_hbm.at[sidx.at[pl.ds(0, sub)]],
                                      buf.at[pl.ds(0, sub)], sem).wait()

        def load_didx(j, didx, sem):
            pltpu.make_async_copy(dst_hbm.at[pl.ds(ebase + j * 128, 128)],
                                  didx, sem).start()

        def wait_didx(didx, sem):
            pltpu.make_async_copy(dst_hbm.at[pl.ds(0, 128)], didx, sem).wait()

        def scat(buf, didx, sem):
            pltpu.make_async_copy(buf, acc_sh.at[didx], sem).start(add=True)

        def wait_scat(buf, didx, sem):
            pltpu.make_async_copy(buf, acc_sh.at[didx], sem).wait()

        load_didx(0, didx_a, sd_a)
        gather(0, buf_a, sg_a)

        def pair(t, _):
            wait_gather(buf_a, sg_a)
            wait_didx(didx_a, sd_a)
            @pl.when(t > 0)
            def _():
                wait_scat(buf_b, didx_b, ss_b)
            load_didx(2 * t + 1, didx_b, sd_b)
            gather(2 * t + 1, buf_b, sg_b)
            scat(buf_a, didx_a, ss_a)
            wait_gather(buf_b, sg_b)
            wait_didx(didx_b, sd_b)
            wait_scat(buf_a, didx_a, ss_a)
            @pl.when(t < pairs - 1)
            def _():
                load_didx(2 * t + 2, didx_a, sd_a)
                gather(2 * t + 2, buf_a, sg_a)
            scat(buf_b, didx_b, ss_b)
            return ()

        lax.fori_loop(0, pairs, pair, ())
        wait_scat(buf_b, didx_b, ss_b)
        plsc.subcore_barrier()

        pltpu.sync_copy(acc_sh.at[pl.ds(row0, span)],
                        acc_hbm.at[c, pl.ds(row0, span)])

    return agg_kernel


def _tc_pool(acc2, degp, ctb2, b2, n, n_types, blk):
    dh = acc2.shape[2]
    d = NC * dh
    nk = n // blk

    def body(a_ref, degp_ref, t_ref, b_ref, o_ref, acc, cnt):
        k = pl.program_id(0)
        deg = degp_ref[0][:, 0:1] + degp_ref[1][:, 0:1] + 1.0
        dinv = jnp.where(deg > 0, lax.rsqrt(deg), 0.0)
        a = jnp.concatenate([a_ref[0], a_ref[1]], axis=1)
        r = jnp.maximum(a * dinv + b_ref[...], 0.0)
        tids = lax.broadcasted_iota(jnp.int32, (blk, n_types), 1)
        m = (t_ref[...] == tids).astype(jnp.float32)
        part = lax.dot_general(m, r, (((0,), (0,)), ((), ())),
                               preferred_element_type=jnp.float32,
                               precision=lax.Precision.HIGHEST)
        cpart = lax.dot_general(m, jnp.ones((blk, 1), jnp.float32),
                                (((0,), (0,)), ((), ())),
                                preferred_element_type=jnp.float32,
                                precision=lax.Precision.HIGHEST)

        @pl.when(k == 0)
        def _():
            acc[...] = jnp.zeros_like(acc)
            cnt[...] = jnp.zeros_like(cnt)

        acc[...] += part
        cnt[...] += cpart

        @pl.when(k == nk - 1)
        def _():
            o_ref[...] = acc[...] / jnp.maximum(cnt[...], 1.0)

    return pl.pallas_call(
        body,
        grid=(nk,),
        in_specs=[
            pl.BlockSpec((NC, blk, dh), lambda i: (0, i, 0)),
            pl.BlockSpec((NC, blk, 1), lambda i: (0, i, 0)),
            pl.BlockSpec((blk, 1), lambda i: (i, 0)),
            pl.BlockSpec((1, d), lambda i: (0, 0)),
        ],
        out_specs=pl.BlockSpec((n_types, d), lambda i: (0, 0)),
        out_shape=jax.ShapeDtypeStruct((n_types, d), jnp.float32),
        scratch_shapes=[
            pltpu.VMEM((n_types, d), jnp.float32),
            pltpu.VMEM((n_types, 1), jnp.float32),
        ],
    )(acc2, degp, ctb2, b2)


def kernel(x, edge_index, batch, cell_type_batch, W, b):
    n, d_in = x.shape
    d_out = W.shape[1]
    dh = d_out // NC
    e = edge_index.shape[1]
    n_types = 100

    span = _node_span(n)
    n_pad = span * NS
    chunk_e = 128 * NC * NS
    e_pad = (e + chunk_e - 1) // chunk_e * chunk_e
    n_chunks_pad = e_pad // 128

    src = edge_index[0]
    dst = edge_index[1]
    src_p = jnp.concatenate([src, jnp.zeros((e_pad - e,), jnp.int32)])
    dst_p = jnp.concatenate([dst, jnp.full((e_pad - e,), n_pad - 1, jnp.int32)])
    src2 = jnp.stack([src_p, src_p + n_pad])
    dst3 = dst_p.reshape(n_chunks_pad, 1, 128)

    degp2 = _make_deg_kernel(n_pad, n_chunks_pad)(dst3)
    degp = degp2.reshape(NC, n_pad, 1)

    g2 = _tc_scale_matmul(x, W, degp, n_pad, blk=1000)
    g_flat = g2.reshape(NC * n_pad, dh)
    acc2 = _make_agg_kernel(n_pad, n_chunks_pad, dh)(g_flat, src2, dst_p)

    pooled = _tc_pool(acc2, degp,
                      cell_type_batch.reshape(n, 1).astype(jnp.int32),
                      b.reshape(1, d_out), n, n_types, blk=1000)
    return pooled
